# baseline (speedup 1.0000x reference)
import functools

import jax
import jax.numpy as jnp
from jax import lax
from jax.experimental import pallas as pl
from jax.experimental.pallas import tpu as pltpu
from jax.experimental.pallas import tpu_sc as plsc

_NUM_POIS = 1000000
_EMBED_DIM = 64
_BATCH = 4096
_HIST = 50

_NC = 2
_NS = 16
_NW = _NC * _NS
_TOTAL = _BATCH * _HIST
_B_PER_W = _TOTAL // _NW
_CHUNK = 128
_N_CHUNKS = _B_PER_W // _CHUNK

_mesh = plsc.VectorSubcoreMesh(core_axis_name="c", subcore_axis_name="s")


@functools.partial(
    pl.kernel,
    mesh=_mesh,
    out_type=jax.ShapeDtypeStruct((_TOTAL, _EMBED_DIM), jnp.float32),
    scratch_types=[
        pltpu.VMEM((_N_CHUNKS, _CHUNK), jnp.int32),
        pltpu.VMEM((_CHUNK, _EMBED_DIM), jnp.float32),
        pltpu.SemaphoreType.DMA,
    ],
    compiler_params=pltpu.CompilerParams(use_tc_tiling_on_sc=False),
)
def _sc_gather(idx_hbm, table_hbm, out_hbm, idx_v, rows_v, sem):
    wid = lax.axis_index("s") * _NC + lax.axis_index("c")
    base = wid * _B_PER_W
    pltpu.sync_copy(idx_hbm.at[wid], idx_v)

    def step(j, carry):
        pltpu.async_copy(table_hbm.at[idx_v.at[j]], rows_v, sem).wait()
        off = pl.multiple_of(base + j * _CHUNK, _CHUNK)
        pltpu.sync_copy(rows_v, out_hbm.at[pl.ds(off, _CHUNK)])
        return carry

    lax.fori_loop(0, _N_CHUNKS, step, 0)


def kernel(poi_idx, geo_embedding_weight):
    idx = poi_idx.astype(jnp.int32).reshape(_NW, _N_CHUNKS, _CHUNK)
    out = _sc_gather(idx, geo_embedding_weight)
    return out.reshape(_BATCH, _HIST, _EMBED_DIM)


# speedup vs baseline: 1.0466x; 1.0466x over previous
import functools

import jax
import jax.numpy as jnp
from jax import lax
from jax.experimental import pallas as pl
from jax.experimental.pallas import tpu as pltpu
from jax.experimental.pallas import tpu_sc as plsc

_NUM_POIS = 1000000
_EMBED_DIM = 64
_BATCH = 4096
_HIST = 50

_NC = 2
_NS = 16
_NW = _NC * _NS
_TOTAL = _BATCH * _HIST
_B_PER_W = _TOTAL // _NW
_CHUNK = 128
_N_CHUNKS = _B_PER_W // _CHUNK
_NBUF = 5

_mesh = plsc.VectorSubcoreMesh(core_axis_name="c", subcore_axis_name="s")


@functools.partial(
    pl.kernel,
    mesh=_mesh,
    out_type=jax.ShapeDtypeStruct((_TOTAL, _EMBED_DIM), jnp.float32),
    scratch_types=[
        pltpu.VMEM((_N_CHUNKS, _CHUNK), jnp.int32),
        pltpu.VMEM((_NBUF, _CHUNK, _EMBED_DIM), jnp.float32),
        pltpu.SemaphoreType.DMA((_NBUF,)),
        pltpu.SemaphoreType.DMA((_NBUF,)),
    ],
    compiler_params=pltpu.CompilerParams(use_tc_tiling_on_sc=False),
)
def _sc_gather(idx_hbm, table_hbm, out_hbm, idx_v, rows_v, gsem, wsem):
    wid = lax.axis_index("s") * _NC + lax.axis_index("c")
    base = wid * _B_PER_W
    pltpu.sync_copy(idx_hbm.at[wid], idx_v)

    def gather(j, b):
        pltpu.async_copy(table_hbm.at[idx_v.at[j]], rows_v.at[b], gsem.at[b])

    def writeback(j, b):
        off = pl.multiple_of(base + j * _CHUNK, _CHUNK)
        return pltpu.async_copy(rows_v.at[b], out_hbm.at[pl.ds(off, _CHUNK)],
                                wsem.at[b])

    for b in range(_NBUF):
        gather(b, b)

    def step(i, carry):
        g = i * _NBUF
        for b in range(_NBUF):
            j = g + b
            pltpu.make_async_copy(table_hbm.at[idx_v.at[j]], rows_v.at[b],
                                  gsem.at[b]).wait()
            writeback(j, b).wait()
            gather(j + _NBUF, b)
        return carry

    lax.fori_loop(0, _N_CHUNKS // _NBUF - 1, step, 0)

    handles = []
    for b in range(_NBUF):
        j = _N_CHUNKS - _NBUF + b
        pltpu.make_async_copy(table_hbm.at[idx_v.at[j]], rows_v.at[b],
                              gsem.at[b]).wait()
        handles.append(writeback(j, b))
    for h in handles:
        h.wait()


def kernel(poi_idx, geo_embedding_weight):
    idx = poi_idx.astype(jnp.int32).reshape(_NW, _N_CHUNKS, _CHUNK)
    out = _sc_gather(idx, geo_embedding_weight)
    return out.reshape(_BATCH, _HIST, _EMBED_DIM)
